# initial kernel scaffold (unmeasured)
import jax
import jax.numpy as jnp
from jax import lax
from jax.experimental import pallas as pl
from jax.experimental.pallas import tpu as pltpu

N_DEV = 4
N_HOPS = N_DEV - 1
NC = 4
COMM_DTYPE = jnp.bfloat16


def kernel(x, w_mat, scale_x, scale_w):
    m_total, k = x.shape
    _, n = w_mat.shape
    mb = m_total // N_DEV
    wn = n // NC

    def body(x_ref, w_ref, sx_ref, sw_ref, out_ref,
             x16_ref, w_stage_ref, w16_ref, comm_ref,
             send_sems, recv_sems, copy_sem):
        d = lax.axis_index("i")
        left = lax.rem(d + N_DEV - 1, N_DEV)
        right = lax.rem(d + 1, N_DEV)

        barrier = pltpu.get_barrier_semaphore()
        for nbr in (left, right):
            pl.semaphore_signal(barrier, inc=1, device_id=(nbr,),
                                device_id_type=pl.DeviceIdType.MESH)
        pl.semaphore_wait(barrier, 2)

        x16_ref[...] = x_ref[...].astype(jnp.bfloat16)

        def partial(c):
            blk = x16_ref[pl.ds(c * mb, mb), :]
            return jnp.dot(blk, w16_ref[...], preferred_element_type=jnp.float32)

        for j in range(NC):
            cp = pltpu.make_async_copy(
                w_ref.at[:, pl.ds(j * wn, wn)], w_stage_ref, copy_sem)
            cp.start()
            cp.wait()
            w16_ref[...] = w_stage_ref[...].astype(jnp.bfloat16)

            comm_ref[0, :, :] = partial(left).astype(COMM_DTYPE)

            for h in range(N_HOPS):
                s_slot = h % 3
                r_slot = (h + 1) % 3
                rdma = pltpu.make_async_remote_copy(
                    src_ref=comm_ref.at[s_slot],
                    dst_ref=comm_ref.at[r_slot],
                    send_sem=send_sems.at[j, h],
                    recv_sem=recv_sems.at[j, h],
                    device_id=(right,),
                    device_id_type=pl.DeviceIdType.MESH,
                )
                rdma.start()
                rdma.wait()

                c = lax.rem(d + 2 * N_DEV - 2 - h, N_DEV)
                acc = comm_ref[r_slot, :, :].astype(jnp.float32) + partial(c)
                if h < N_HOPS - 1:
                    comm_ref[r_slot, :, :] = acc.astype(COMM_DTYPE)
                else:
                    scale = sx_ref[0] * sw_ref[0]
                    out_ref[:, j * wn:(j + 1) * wn] = acc * scale

    out_shape = jax.ShapeDtypeStruct((mb, n), jnp.float32)
    return pl.pallas_call(
        body,
        out_shape=out_shape,
        in_specs=[
            pl.BlockSpec(memory_space=pltpu.VMEM),
            pl.BlockSpec(memory_space=pltpu.ANY),
            pl.BlockSpec(memory_space=pltpu.SMEM),
            pl.BlockSpec(memory_space=pltpu.SMEM),
        ],
        out_specs=pl.BlockSpec(memory_space=pltpu.VMEM),
        scratch_shapes=[
            pltpu.VMEM((m_total, k), jnp.bfloat16),
            pltpu.VMEM((k, wn), w_mat.dtype),
            pltpu.VMEM((k, wn), jnp.bfloat16),
            pltpu.VMEM((3, mb, wn), COMM_DTYPE),
            pltpu.SemaphoreType.DMA((NC, N_HOPS)),
            pltpu.SemaphoreType.DMA((NC, N_HOPS)),
            pltpu.SemaphoreType.DMA,
        ],
        compiler_params=pltpu.CompilerParams(collective_id=0),
    )(x, w_mat, scale_x, scale_w)


# baseline (device time: 726844 ns/iter reference)
import jax
import jax.numpy as jnp
from jax import lax
from jax.experimental import pallas as pl
from jax.experimental.pallas import tpu as pltpu

N_DEV = 4
N_HOPS = N_DEV - 1
NC = 8
COMM_DTYPE = jnp.bfloat16


def kernel(x, w_mat, scale_x, scale_w):
    m_total, k = x.shape
    _, n = w_mat.shape
    mb = m_total // N_DEV
    wn = n // NC

    def body(x_ref, w_ref, sx_ref, sw_ref, out_ref,
             w_stage_ref, w16_ref, comm_ref, y_stage_ref,
             send_sems, recv_sems, copy_sem, out_sem):
        d = lax.axis_index("i")
        left = lax.rem(d + N_DEV - 1, N_DEV)
        right = lax.rem(d + 1, N_DEV)

        barrier = pltpu.get_barrier_semaphore()
        for nbr in (left, right):
            pl.semaphore_signal(barrier, inc=1, device_id=(nbr,),
                                device_id_type=pl.DeviceIdType.MESH)
        pl.semaphore_wait(barrier, 2)

        def partial(c):
            blk = x_ref[pl.ds(c * mb, mb), :].astype(jnp.bfloat16)
            return jnp.dot(blk, w16_ref[...], preferred_element_type=jnp.float32)

        for j in range(NC):
            cp = pltpu.make_async_copy(
                w_ref.at[:, pl.ds(j * wn, wn)], w_stage_ref, copy_sem)
            cp.start()
            cp.wait()
            w16_ref[...] = w_stage_ref[...].astype(jnp.bfloat16)

            comm_ref[0, :, :] = partial(left).astype(COMM_DTYPE)

            for h in range(N_HOPS):
                s_slot = h % 3
                r_slot = (h + 1) % 3
                rdma = pltpu.make_async_remote_copy(
                    src_ref=comm_ref.at[s_slot],
                    dst_ref=comm_ref.at[r_slot],
                    send_sem=send_sems.at[j, h],
                    recv_sem=recv_sems.at[j, h],
                    device_id=(right,),
                    device_id_type=pl.DeviceIdType.MESH,
                )
                rdma.start()
                rdma.wait()

                c = lax.rem(d + 2 * N_DEV - 2 - h, N_DEV)
                acc = comm_ref[r_slot, :, :].astype(jnp.float32) + partial(c)
                if h < N_HOPS - 1:
                    comm_ref[r_slot, :, :] = acc.astype(COMM_DTYPE)
                else:
                    scale = sx_ref[0] * sw_ref[0]
                    y_stage_ref[...] = acc * scale
                    ocp = pltpu.make_async_copy(
                        y_stage_ref, out_ref.at[:, pl.ds(j * wn, wn)], out_sem)
                    ocp.start()
                    ocp.wait()

    out_shape = jax.ShapeDtypeStruct((mb, n), jnp.float32)
    return pl.pallas_call(
        body,
        out_shape=out_shape,
        in_specs=[
            pl.BlockSpec(memory_space=pltpu.VMEM),
            pl.BlockSpec(memory_space=pl.ANY),
            pl.BlockSpec(memory_space=pltpu.SMEM),
            pl.BlockSpec(memory_space=pltpu.SMEM),
        ],
        out_specs=pl.BlockSpec(memory_space=pl.ANY),
        scratch_shapes=[
            pltpu.VMEM((k, wn), w_mat.dtype),
            pltpu.VMEM((k, wn), jnp.bfloat16),
            pltpu.VMEM((3, mb, wn), COMM_DTYPE),
            pltpu.VMEM((mb, wn), jnp.float32),
            pltpu.SemaphoreType.DMA((NC, N_HOPS)),
            pltpu.SemaphoreType.DMA((NC, N_HOPS)),
            pltpu.SemaphoreType.DMA,
            pltpu.SemaphoreType.DMA,
        ],
        compiler_params=pltpu.CompilerParams(collective_id=0),
    )(x, w_mat, scale_x, scale_w)


# device time: 384281 ns/iter; 1.8914x vs baseline; 1.8914x over previous
import jax
import jax.numpy as jnp
from jax import lax
from jax.experimental import pallas as pl
from jax.experimental.pallas import tpu as pltpu

N_DEV = 4
N_HOPS = N_DEV - 1
NPAIR = 4
COMM_DTYPE = jnp.bfloat16


def kernel(x, w_mat, scale_x, scale_w):
    m_total, k = x.shape
    _, n = w_mat.shape
    mb = m_total // N_DEV
    wn = n // (2 * NPAIR)

    def body(x_ref, w_ref, sx_ref, sw_ref, out_ref,
             w_stage_ref, w16_ref, commr_ref, comml_ref, y_stage_ref,
             sendr, recvr, sendl, recvl, w_sem, out_sem):
        d = lax.axis_index("i")
        left = lax.rem(d + N_DEV - 1, N_DEV)
        right = lax.rem(d + 1, N_DEV)
        scale = sx_ref[0] * sw_ref[0]

        barrier = pltpu.get_barrier_semaphore()
        for nbr in (left, right):
            pl.semaphore_signal(barrier, inc=1, device_id=(nbr,),
                                device_id_type=pl.DeviceIdType.MESH)
        pl.semaphore_wait(barrier, 2)

        def partial(c, half):
            blk = x_ref[pl.ds(c * mb, mb), :].astype(jnp.bfloat16)
            wblk = w16_ref[:, half * wn:(half + 1) * wn]
            return jnp.dot(blk, wblk, preferred_element_type=jnp.float32)

        for p in range(NPAIR):
            cp = pltpu.make_async_copy(
                w_ref.at[:, pl.ds(p * 2 * wn, 2 * wn)], w_stage_ref, w_sem)
            cp.start()
            cp.wait()
            w16_ref[...] = w_stage_ref[...].astype(jnp.bfloat16)

            commr_ref[0, :, :] = partial(left, 0).astype(COMM_DTYPE)
            comml_ref[0, :, :] = partial(right, 1).astype(COMM_DTYPE)

            for h in range(N_HOPS):
                s_slot = h % 2
                r_slot = (h + 1) % 2
                rdma_r = pltpu.make_async_remote_copy(
                    src_ref=commr_ref.at[s_slot],
                    dst_ref=commr_ref.at[r_slot],
                    send_sem=sendr.at[p, h],
                    recv_sem=recvr.at[p, h],
                    device_id=(right,),
                    device_id_type=pl.DeviceIdType.MESH,
                )
                rdma_l = pltpu.make_async_remote_copy(
                    src_ref=comml_ref.at[s_slot],
                    dst_ref=comml_ref.at[r_slot],
                    send_sem=sendl.at[p, h],
                    recv_sem=recvl.at[p, h],
                    device_id=(left,),
                    device_id_type=pl.DeviceIdType.MESH,
                )
                rdma_r.start()
                rdma_l.start()

                cr = lax.rem(d + 2 * N_DEV - 2 - h, N_DEV)
                cl = lax.rem(d + 2 + h, N_DEV)
                part_r = partial(cr, 0)
                part_l = partial(cl, 1)

                rdma_r.wait()
                rdma_l.wait()
                acc_r = commr_ref[r_slot, :, :].astype(jnp.float32) + part_r
                acc_l = comml_ref[r_slot, :, :].astype(jnp.float32) + part_l
                if h < N_HOPS - 1:
                    commr_ref[r_slot, :, :] = acc_r.astype(COMM_DTYPE)
                    comml_ref[r_slot, :, :] = acc_l.astype(COMM_DTYPE)
                else:
                    y_stage_ref[:, :wn] = acc_r * scale
                    y_stage_ref[:, wn:] = acc_l * scale
                    ocp = pltpu.make_async_copy(
                        y_stage_ref, out_ref.at[:, pl.ds(p * 2 * wn, 2 * wn)],
                        out_sem)
                    ocp.start()
                    ocp.wait()

    out_shape = jax.ShapeDtypeStruct((mb, n), jnp.float32)
    return pl.pallas_call(
        body,
        out_shape=out_shape,
        in_specs=[
            pl.BlockSpec(memory_space=pltpu.VMEM),
            pl.BlockSpec(memory_space=pl.ANY),
            pl.BlockSpec(memory_space=pltpu.SMEM),
            pl.BlockSpec(memory_space=pltpu.SMEM),
        ],
        out_specs=pl.BlockSpec(memory_space=pl.ANY),
        scratch_shapes=[
            pltpu.VMEM((k, 2 * wn), w_mat.dtype),
            pltpu.VMEM((k, 2 * wn), jnp.bfloat16),
            pltpu.VMEM((2, mb, wn), COMM_DTYPE),
            pltpu.VMEM((2, mb, wn), COMM_DTYPE),
            pltpu.VMEM((mb, 2 * wn), jnp.float32),
            pltpu.SemaphoreType.DMA((NPAIR, N_HOPS)),
            pltpu.SemaphoreType.DMA((NPAIR, N_HOPS)),
            pltpu.SemaphoreType.DMA((NPAIR, N_HOPS)),
            pltpu.SemaphoreType.DMA((NPAIR, N_HOPS)),
            pltpu.SemaphoreType.DMA,
            pltpu.SemaphoreType.DMA,
        ],
        compiler_params=pltpu.CompilerParams(
            collective_id=0, vmem_limit_bytes=60 * 1024 * 1024),
    )(x, w_mat, scale_x, scale_w)
